# Initial kernel scaffold; baseline (speedup 1.0000x reference)
import jax
import jax.numpy as jnp
from jax.experimental import pallas as pl


def kernel(bags, W_enc, b_enc, Wa1, ba1, Wa2, ba2, g_r, be_r, W_rh, b_rh, Ws1, bs1, Ws2, bs2, g_s, be_s, W_sh, b_sh):
    raise NotImplementedError("write your pallas kernel here")


# speedup vs baseline: 2.0301x; 2.0301x over previous
import numpy as np
import jax
import jax.numpy as jnp
from jax.experimental import pallas as pl
from jax.experimental.pallas import tpu as pltpu

K = 10
EPS = 1e-5
KM_ITERS = 5
PTILE = 1024
_HI = jax.lax.Precision.HIGHEST
_NEG = -1e30


def _bag_kernel(bags_ref, W_enc_ref, b_enc_ref, Wa1_ref, ba1_ref, Wa2_ref,
                ba2_ref, g_r_ref, be_r_ref, W_rh_ref, b_rh_ref, Ws1_ref,
                bs1_ref, Ws2_ref, bs2_ref, g_s_ref, be_s_ref, W_shp_ref,
                b_shp_ref, out_ref, emb_ref, a_ref):
    pt = pl.program_id(1)
    npt = pl.num_programs(1)
    P = emb_ref.shape[0]

    tile = bags_ref[0]
    tsz = tile.shape[0]
    e = jnp.maximum(
        jnp.dot(tile, W_enc_ref[...], preferred_element_type=jnp.float32)
        + b_enc_ref[...], 0.0)
    h = jnp.tanh(
        jnp.dot(e, Wa1_ref[...], preferred_element_type=jnp.float32)
        + ba1_ref[...])
    at = (jnp.dot(h, Wa2_ref[...], preferred_element_type=jnp.float32)
          + ba2_ref[...])
    emb_ref[pl.ds(pt * tsz, tsz), :] = e
    a_ref[pl.ds(pt * tsz, tsz), :] = at

    @pl.when(pt == npt - 1)
    def _phase2():
        emb = emb_ref[...]
        a = a_ref[...]

        xsq = jnp.sum(emb * emb, axis=1, keepdims=True)
        iota_k = jax.lax.broadcasted_iota(jnp.int32, (P, K), 1)

        idx = np.linspace(0, P - 1, K).astype(np.int32)
        cent = jnp.concatenate([emb[i:i + 1, :] for i in idx], axis=0)

        def assign(cent):
            csq = jnp.sum(cent * cent, axis=1)
            xc = jax.lax.dot_general(emb, cent, (((1,), (1,)), ((), ())),
                                     precision=_HI,
                                     preferred_element_type=jnp.float32)
            d = xsq - 2.0 * xc + csq[None, :]
            dmin = d[:, 0:1]
            amin = jnp.zeros((P, 1), jnp.int32)
            for k in range(1, K):
                dk = d[:, k:k + 1]
                lt = dk < dmin
                amin = jnp.where(lt, k, amin)
                dmin = jnp.where(lt, dk, dmin)
            return (amin == iota_k).astype(jnp.float32)

        cent_c = cent
        for _ in range(KM_ITERS):
            onehot = assign(cent_c)
            cnt = jnp.sum(onehot, axis=0)
            s = jax.lax.dot_general(onehot, emb, (((0,), (0,)), ((), ())),
                                    precision=_HI,
                                    preferred_element_type=jnp.float32)
            cent_c = s / jnp.maximum(cnt, 1.0)[:, None]
        onehot = assign(cent_c)

        A = jnp.where(onehot > 0.0, a, _NEG)
        m = jnp.max(A, axis=0)
        E = onehot * jnp.exp(A - m[None, :])
        sseg = jnp.sum(E, axis=0)
        W = E / jnp.maximum(sseg, 1e-12)[None, :]

        region = jax.lax.dot_general(W, emb, (((0,), (0,)), ((), ())),
                                     precision=_HI,
                                     preferred_element_type=jnp.float32)
        reg_bn = (region * (1.0 / np.sqrt(1.0 + EPS)) * g_r_ref[...]
                  + be_r_ref[...])
        reg_out = (jnp.dot(reg_bn, W_rh_ref[...],
                           preferred_element_type=jnp.float32)
                   + b_rh_ref[...])

        hs = jnp.tanh(
            jnp.dot(reg_out, Ws1_ref[...], preferred_element_type=jnp.float32)
            + bs1_ref[...])
        sa = (jnp.dot(hs, Ws2_ref[...], preferred_element_type=jnp.float32)
              + bs2_ref[...])
        aw = jnp.exp(sa - jnp.max(sa))
        aw = aw / jnp.sum(aw)
        slide = jnp.sum(aw * reg_out, axis=0, keepdims=True)
        slide_bn = (slide * (1.0 / np.sqrt(1.0 + EPS)) * g_s_ref[...]
                    + be_s_ref[...])
        out_ref[0] = (jnp.dot(slide_bn, W_shp_ref[...],
                              preferred_element_type=jnp.float32)
                      + b_shp_ref[...])


def kernel(bags, W_enc, b_enc, Wa1, ba1, Wa2, ba2, g_r, be_r, W_rh, b_rh,
           Ws1, bs1, Ws2, bs2, g_s, be_s, W_sh, b_sh):
    B, P, F = bags.shape
    Z = W_enc.shape[1]
    NOUT = W_sh.shape[1]
    OPAD = 128
    ptile = min(PTILE, P)
    npt = P // ptile
    assert P % ptile == 0

    W_shp = jnp.zeros((Z, OPAD), jnp.float32).at[:, :NOUT].set(W_sh)
    b_shp = jnp.zeros((1, OPAD), jnp.float32).at[:, :NOUT].set(b_sh[None, :])

    full = lambda *shape: pl.BlockSpec(shape, lambda b, pt: tuple(0 for _ in shape))
    out = pl.pallas_call(
        _bag_kernel,
        grid=(B, npt),
        in_specs=[
            pl.BlockSpec((1, ptile, F), lambda b, pt: (b, pt, 0)),
            full(F, Z), full(1, Z),
            full(Z, Z), full(1, Z),
            full(Z, 1), full(1, 1),
            full(1, Z), full(1, Z),
            full(Z, Z), full(1, Z),
            full(Z, Z), full(1, Z),
            full(Z, 1), full(1, 1),
            full(1, Z), full(1, Z),
            full(Z, OPAD), full(1, OPAD),
        ],
        out_specs=pl.BlockSpec((1, 1, OPAD), lambda b, pt: (b, 0, 0)),
        out_shape=jax.ShapeDtypeStruct((B, 1, OPAD), jnp.float32),
        scratch_shapes=[
            pltpu.VMEM((P, Z), jnp.float32),
            pltpu.VMEM((P, 1), jnp.float32),
        ],
        compiler_params=pltpu.CompilerParams(
            dimension_semantics=("arbitrary", "arbitrary"),
        ),
    )(bags, W_enc, b_enc[None, :], Wa1, ba1[None, :], Wa2, ba2[None, :],
      g_r[None, :], be_r[None, :], W_rh, b_rh[None, :], Ws1, bs1[None, :],
      Ws2, bs2[None, :], g_s[None, :], be_s[None, :], W_shp, b_shp)
    return out[:, 0, :NOUT]


# speedup vs baseline: 6.8421x; 3.3704x over previous
import numpy as np
import jax
import jax.numpy as jnp
from jax.experimental import pallas as pl
from jax.experimental.pallas import tpu as pltpu

K = 10
EPS = 1e-5
KM_ITERS = 5
PTILE = 1024
_HI = jax.lax.Precision.HIGHEST
_NEG = -1e30


def _bag_kernel(bags_ref, W_enc_ref, b_enc_ref, Wa1_ref, ba1_ref, Wa2_ref,
                ba2_ref, g_r_ref, be_r_ref, W_rh_ref, b_rh_ref, Ws1_ref,
                bs1_ref, Ws2_ref, bs2_ref, g_s_ref, be_s_ref, W_shp_ref,
                b_shp_ref, out_ref, emb_ref, a_ref):
    pt = pl.program_id(1)
    npt = pl.num_programs(1)
    P = emb_ref.shape[0]

    tile = bags_ref[0]
    tsz = tile.shape[0]
    e = jnp.maximum(
        jnp.dot(tile, W_enc_ref[...], preferred_element_type=jnp.float32)
        + b_enc_ref[...], 0.0)
    h = jnp.tanh(
        jnp.dot(e, Wa1_ref[...], preferred_element_type=jnp.float32)
        + ba1_ref[...])
    at = (jax.lax.dot_general(Wa2_ref[...], h, (((0,), (1,)), ((), ())),
                              preferred_element_type=jnp.float32)
          + ba2_ref[...])
    emb_ref[pl.ds(pt * tsz, tsz), :] = e
    a_ref[:, pl.ds(pt * tsz, tsz)] = at

    @pl.when(pt == npt - 1)
    def _phase2():
        emb = emb_ref[...]
        a_row = a_ref[...]

        iota_kp = jax.lax.broadcasted_iota(jnp.int32, (K, P), 0)

        idx = np.linspace(0, P - 1, K).astype(np.int32)
        cent = jnp.concatenate([emb[i:i + 1, :] for i in idx], axis=0)

        def assign(cent):
            csq = jnp.sum(cent * cent, axis=1, keepdims=True)
            xc = jax.lax.dot_general(cent, emb, (((1,), (1,)), ((), ())),
                                     precision=_HI,
                                     preferred_element_type=jnp.float32)
            d = csq - 2.0 * xc
            dmin = jnp.min(d, axis=0, keepdims=True)
            amin = jnp.min(jnp.where(d == dmin, iota_kp, K),
                           axis=0, keepdims=True)
            return (iota_kp == amin).astype(jnp.float32)

        cent_c = cent
        for _ in range(KM_ITERS):
            onehot = assign(cent_c)
            cnt = jnp.sum(onehot, axis=1, keepdims=True)
            s = jnp.dot(onehot, emb, precision=_HI,
                        preferred_element_type=jnp.float32)
            cent_c = s / jnp.maximum(cnt, 1.0)
        onehot = assign(cent_c)

        A = jnp.where(onehot > 0.0, a_row, _NEG)
        m = jnp.max(A, axis=1, keepdims=True)
        E = onehot * jnp.exp(A - m)
        sseg = jnp.sum(E, axis=1, keepdims=True)
        W = E / jnp.maximum(sseg, 1e-12)

        region = jnp.dot(W, emb, precision=_HI,
                         preferred_element_type=jnp.float32)
        reg_bn = (region * (1.0 / np.sqrt(1.0 + EPS)) * g_r_ref[...]
                  + be_r_ref[...])
        reg_out = (jnp.dot(reg_bn, W_rh_ref[...],
                           preferred_element_type=jnp.float32)
                   + b_rh_ref[...])

        hs = jnp.tanh(
            jnp.dot(reg_out, Ws1_ref[...], preferred_element_type=jnp.float32)
            + bs1_ref[...])
        sa = (jnp.dot(hs, Ws2_ref[...], preferred_element_type=jnp.float32)
              + bs2_ref[...])
        aw = jnp.exp(sa - jnp.max(sa))
        aw = aw / jnp.sum(aw)
        slide = jnp.sum(aw * reg_out, axis=0, keepdims=True)
        slide_bn = (slide * (1.0 / np.sqrt(1.0 + EPS)) * g_s_ref[...]
                    + be_s_ref[...])
        out_ref[0] = (jnp.dot(slide_bn, W_shp_ref[...],
                              preferred_element_type=jnp.float32)
                      + b_shp_ref[...])


def kernel(bags, W_enc, b_enc, Wa1, ba1, Wa2, ba2, g_r, be_r, W_rh, b_rh,
           Ws1, bs1, Ws2, bs2, g_s, be_s, W_sh, b_sh):
    B, P, F = bags.shape
    Z = W_enc.shape[1]
    NOUT = W_sh.shape[1]
    OPAD = 128
    ptile = min(PTILE, P)
    npt = P // ptile
    assert P % ptile == 0

    W_shp = jnp.zeros((Z, OPAD), jnp.float32).at[:, :NOUT].set(W_sh)
    b_shp = jnp.zeros((1, OPAD), jnp.float32).at[:, :NOUT].set(b_sh[None, :])

    full = lambda *shape: pl.BlockSpec(shape, lambda b, pt: tuple(0 for _ in shape))
    out = pl.pallas_call(
        _bag_kernel,
        grid=(B, npt),
        in_specs=[
            pl.BlockSpec((1, ptile, F), lambda b, pt: (b, pt, 0)),
            full(F, Z), full(1, Z),
            full(Z, Z), full(1, Z),
            full(Z, 1), full(1, 1),
            full(1, Z), full(1, Z),
            full(Z, Z), full(1, Z),
            full(Z, Z), full(1, Z),
            full(Z, 1), full(1, 1),
            full(1, Z), full(1, Z),
            full(Z, OPAD), full(1, OPAD),
        ],
        out_specs=pl.BlockSpec((1, 1, OPAD), lambda b, pt: (b, 0, 0)),
        out_shape=jax.ShapeDtypeStruct((B, 1, OPAD), jnp.float32),
        scratch_shapes=[
            pltpu.VMEM((P, Z), jnp.float32),
            pltpu.VMEM((1, P), jnp.float32),
        ],
        compiler_params=pltpu.CompilerParams(
            dimension_semantics=("arbitrary", "arbitrary"),
        ),
    )(bags, W_enc, b_enc[None, :], Wa1, ba1[None, :], Wa2, ba2[None, :],
      g_r[None, :], be_r[None, :], W_rh, b_rh[None, :], Ws1, bs1[None, :],
      Ws2, bs2[None, :], g_s[None, :], be_s[None, :], W_shp, b_shp)
    return out[:, 0, :NOUT]


# speedup vs baseline: 18.5456x; 2.7105x over previous
import numpy as np
import jax
import jax.numpy as jnp
from jax.experimental import pallas as pl
from jax.experimental.pallas import tpu as pltpu

K = 10
EPS = 1e-5
KM_ITERS = 5
PTILE = 1024
_HI = jax.lax.Precision.HIGHEST
_NEG = -1e30


def _bag_kernel(bags_ref, W_enc_ref, b_enc_ref, Wa1_ref, ba1_ref, Wa2_ref,
                ba2_ref, g_r_ref, be_r_ref, W_rh_ref, b_rh_ref, Ws1_ref,
                bs1_ref, Ws2_ref, bs2_ref, g_s_ref, be_s_ref, W_shp_ref,
                b_shp_ref, out_ref, emb_ref, a_ref):
    pt = pl.program_id(1)
    npt = pl.num_programs(1)
    P = emb_ref.shape[0]

    tile = bags_ref[0]
    tsz = tile.shape[0]
    e = jnp.maximum(
        jnp.dot(tile, W_enc_ref[...], preferred_element_type=jnp.float32)
        + b_enc_ref[...], 0.0)
    h = jnp.tanh(
        jnp.dot(e, Wa1_ref[...], preferred_element_type=jnp.float32)
        + ba1_ref[...])
    at = (jax.lax.dot_general(Wa2_ref[...], h, (((0,), (1,)), ((), ())),
                              preferred_element_type=jnp.float32)
          + ba2_ref[...])
    emb_ref[pl.ds(pt * tsz, tsz), :] = e
    a_ref[:, pl.ds(pt * tsz, tsz)] = at

    @pl.when(pt == npt - 1)
    def _phase2():
        emb = emb_ref[...]
        a_row = a_ref[...]

        iota_kp = jax.lax.broadcasted_iota(jnp.int32, (K, P), 0)

        idx = np.linspace(0, P - 1, K).astype(np.int32)
        cent = jnp.concatenate([emb[i:i + 1, :] for i in idx], axis=0)

        def assign(cent):
            csq = jnp.sum(cent * cent, axis=1, keepdims=True)
            xc = jax.lax.dot_general(cent, emb, (((1,), (1,)), ((), ())),

                                     preferred_element_type=jnp.float32)
            d = csq - 2.0 * xc
            dmin = jnp.min(d, axis=0, keepdims=True)
            amin = jnp.min(jnp.where(d == dmin, iota_kp, K),
                           axis=0, keepdims=True)
            return (iota_kp == amin).astype(jnp.float32)

        cent_c = cent
        for _ in range(KM_ITERS):
            onehot = assign(cent_c)
            cnt = jnp.sum(onehot, axis=1, keepdims=True)
            s = jnp.dot(onehot, emb,
                        preferred_element_type=jnp.float32)
            cent_c = s / jnp.maximum(cnt, 1.0)
        onehot = assign(cent_c)

        A = jnp.where(onehot > 0.0, a_row, _NEG)
        m = jnp.max(A, axis=1, keepdims=True)
        E = onehot * jnp.exp(A - m)
        sseg = jnp.sum(E, axis=1, keepdims=True)
        W = E / jnp.maximum(sseg, 1e-12)

        region = jnp.dot(W, emb,
                         preferred_element_type=jnp.float32)
        reg_bn = (region * (1.0 / np.sqrt(1.0 + EPS)) * g_r_ref[...]
                  + be_r_ref[...])
        reg_out = (jnp.dot(reg_bn, W_rh_ref[...],
                           preferred_element_type=jnp.float32)
                   + b_rh_ref[...])

        hs = jnp.tanh(
            jnp.dot(reg_out, Ws1_ref[...], preferred_element_type=jnp.float32)
            + bs1_ref[...])
        sa = (jnp.dot(hs, Ws2_ref[...], preferred_element_type=jnp.float32)
              + bs2_ref[...])
        aw = jnp.exp(sa - jnp.max(sa))
        aw = aw / jnp.sum(aw)
        slide = jnp.sum(aw * reg_out, axis=0, keepdims=True)
        slide_bn = (slide * (1.0 / np.sqrt(1.0 + EPS)) * g_s_ref[...]
                    + be_s_ref[...])
        out_ref[0] = (jnp.dot(slide_bn, W_shp_ref[...],
                              preferred_element_type=jnp.float32)
                      + b_shp_ref[...])


def kernel(bags, W_enc, b_enc, Wa1, ba1, Wa2, ba2, g_r, be_r, W_rh, b_rh,
           Ws1, bs1, Ws2, bs2, g_s, be_s, W_sh, b_sh):
    B, P, F = bags.shape
    Z = W_enc.shape[1]
    NOUT = W_sh.shape[1]
    OPAD = 128
    ptile = min(PTILE, P)
    npt = P // ptile
    assert P % ptile == 0

    W_shp = jnp.zeros((Z, OPAD), jnp.float32).at[:, :NOUT].set(W_sh)
    b_shp = jnp.zeros((1, OPAD), jnp.float32).at[:, :NOUT].set(b_sh[None, :])

    full = lambda *shape: pl.BlockSpec(shape, lambda b, pt: tuple(0 for _ in shape))
    out = pl.pallas_call(
        _bag_kernel,
        grid=(B, npt),
        in_specs=[
            pl.BlockSpec((1, ptile, F), lambda b, pt: (b, pt, 0)),
            full(F, Z), full(1, Z),
            full(Z, Z), full(1, Z),
            full(Z, 1), full(1, 1),
            full(1, Z), full(1, Z),
            full(Z, Z), full(1, Z),
            full(Z, Z), full(1, Z),
            full(Z, 1), full(1, 1),
            full(1, Z), full(1, Z),
            full(Z, OPAD), full(1, OPAD),
        ],
        out_specs=pl.BlockSpec((1, 1, OPAD), lambda b, pt: (b, 0, 0)),
        out_shape=jax.ShapeDtypeStruct((B, 1, OPAD), jnp.float32),
        scratch_shapes=[
            pltpu.VMEM((P, Z), jnp.float32),
            pltpu.VMEM((1, P), jnp.float32),
        ],
        compiler_params=pltpu.CompilerParams(
            dimension_semantics=("arbitrary", "arbitrary"),
        ),
    )(bags, W_enc, b_enc[None, :], Wa1, ba1[None, :], Wa2, ba2[None, :],
      g_r[None, :], be_r[None, :], W_rh, b_rh[None, :], Ws1, bs1[None, :],
      Ws2, bs2[None, :], g_s[None, :], be_s[None, :], W_shp, b_shp)
    return out[:, 0, :NOUT]
